# baseline (speedup 1.0000x reference)
import functools

import jax
import jax.numpy as jnp
from jax import lax
from jax.experimental import pallas as pl
from jax.experimental.pallas import tpu as pltpu
from jax.experimental.pallas import tpu_sc as plsc

N = 10000
E = 320000
NC = 2
NS = 16
CH = 128
E_PAD = 327680
N_PAD = 10240
TILE_ROWS = N_PAD // NS
ZROWS = 64
DUMMY = N
NBUF = 5

_MESH = dict(core_axis_name="c", subcore_axis_name="s", num_cores=NC,
             num_subcores=NS)


def _zero_fill(buf, nrows, ncols):
    @pl.loop(0, nrows)
    def _(r):
        for j in range(ncols // 16):
            buf[r, pl.ds(j * 16, 16)] = jnp.zeros((16,), jnp.float32)


def _zero_acc(acc, s, zbuf):
    for k in range(TILE_ROWS // ZROWS):
        pltpu.sync_copy(zbuf, acc.at[pl.ds(s * TILE_ROWS + k * ZROWS, ZROWS)])


def _drain_acc(acc, out_hbm, g, s, zbuf):
    for k in range(TILE_ROWS // ZROWS):
        r0 = s * TILE_ROWS + k * ZROWS
        pltpu.sync_copy(acc.at[pl.ds(r0, ZROWS)], zbuf)
        pltpu.sync_copy(zbuf, out_hbm.at[g, pl.ds(r0, ZROWS)])


def _deg_body(dst2d, degp, dst_all, ones, zbuf, acc):
    c = lax.axis_index("c")
    s = lax.axis_index("s")
    rows = (E_PAD // NC // NS) // CH
    row0 = (c * NS + s) * rows

    pltpu.sync_copy(dst2d.at[pl.ds(row0, rows)], dst_all)

    @pl.loop(0, CH)
    def _(r):
        ones[r, :] = jnp.ones((16,), jnp.float32)

    _zero_fill(zbuf, ZROWS, 16)
    _zero_acc(acc, s, zbuf)
    plsc.subcore_barrier()

    @pl.loop(0, rows)
    def _(i):
        pltpu.sync_copy(ones, acc.at[dst_all.at[i]], add=True)

    plsc.subcore_barrier()
    _drain_acc(acc, degp, c, s, zbuf)


def _sc_deg(dst2d):
    return pl.kernel(
        _deg_body,
        out_type=jax.ShapeDtypeStruct((NC, N_PAD, 16), jnp.float32),
        mesh=plsc.VectorSubcoreMesh(**_MESH),
        scratch_types=[
            pltpu.VMEM((E_PAD // NC // NS // CH, CH), jnp.int32),
            pltpu.VMEM((CH, 16), jnp.float32),
            pltpu.VMEM((ZROWS, 16), jnp.float32),
            pltpu.VMEM_SHARED((N_PAD, 16), jnp.float32),
        ],
    )(dst2d)


STAGED = True


def _scatter_body(gpc, ptab, src2d, dst2d, out, sidx, didx, gbuf,
                  zbuf, stbl, acc, isems, gsems, ssems):
    c = lax.axis_index("c")
    s = lax.axis_index("s")
    rows = (E_PAD // NS) // CH
    row0 = s * rows

    def iload(i, b):
        pltpu.async_copy(src2d.at[row0 + i], sidx.at[b], isems[b])
        pltpu.async_copy(dst2d.at[row0 + i], didx.at[b], isems[b])

    def iwait(i, b, goff):
        pltpu.make_async_copy(src2d.at[row0 + i], sidx.at[b],
                              isems[b]).wait()
        pltpu.make_async_copy(dst2d.at[row0 + i], didx.at[b],
                              isems[b]).wait()
        if not STAGED:
            for j in range(CH // 16):
                v = sidx[b, pl.ds(j * 16, 16)]
                sidx[b, pl.ds(j * 16, 16)] = v + goff

    gsrc = stbl if STAGED else ptab

    def gstart(i, b):
        pltpu.async_copy(gsrc.at[sidx.at[b]], gbuf.at[b], gsems[b])

    def gwait(i, b):
        pltpu.make_async_copy(gsrc.at[sidx.at[b]], gbuf.at[b],
                              gsems[b]).wait()

    def sstart(i, b):
        pltpu.async_copy(gbuf.at[b], acc.at[didx.at[b]], ssems[b],
                         add=True)

    def swait(i, b):
        pltpu.make_async_copy(gbuf.at[b], acc.at[didx.at[b]],
                              ssems[b]).wait()

    for phase in range(gpc):
        g = c * gpc + phase
        goff = g * N_PAD
        if STAGED:
            pltpu.async_copy(
                ptab.at[pl.ds(goff + s * TILE_ROWS, TILE_ROWS)],
                stbl.at[pl.ds(s * TILE_ROWS, TILE_ROWS)], gsems[0])
        _zero_fill(zbuf, ZROWS, 64)
        nz = TILE_ROWS // ZROWS
        for k in range(nz):
            pltpu.async_copy(
                zbuf, acc.at[pl.ds(s * TILE_ROWS + k * ZROWS, ZROWS)],
                ssems[0])
        for k in range(nz):
            pltpu.make_async_copy(
                zbuf, acc.at[pl.ds(s * TILE_ROWS + k * ZROWS, ZROWS)],
                ssems[0]).wait()
        if STAGED:
            pltpu.make_async_copy(
                ptab.at[pl.ds(goff + s * TILE_ROWS, TILE_ROWS)],
                stbl.at[pl.ds(s * TILE_ROWS, TILE_ROWS)], gsems[0]).wait()
        plsc.subcore_barrier()

        for b in range(NBUF):
            iload(b, b)
        for b in range(NBUF):
            iwait(b, b, goff)
            gstart(b, b)

        @pl.loop(0, rows, step=NBUF)
        def _(i):
            for b in range(NBUF):
                gwait(i + b, b)
                sstart(i + b, b)
            for b in range(NBUF):
                swait(i + b, b)

                @pl.when(i + b + NBUF < rows)
                def _():
                    iload(i + b + NBUF, b)
                    iwait(i + b + NBUF, b, goff)
                    gstart(i + b + NBUF, b)

        plsc.subcore_barrier()
        _drain_acc(acc, out, g, s, zbuf)
        if phase + 1 < gpc:
            plsc.subcore_barrier()


def _sc_scatter(ptab, src2d, dst2d, gpc):
    body = functools.partial(_scatter_body, gpc)
    return pl.kernel(
        body,
        out_type=jax.ShapeDtypeStruct((NC * gpc, N_PAD, 64), jnp.float32),
        mesh=plsc.VectorSubcoreMesh(**_MESH),
        scratch_types=[
            pltpu.VMEM((NBUF, CH), jnp.int32),
            pltpu.VMEM((NBUF, CH), jnp.int32),
            pltpu.VMEM((NBUF, CH, 64), jnp.float32),
            pltpu.VMEM((ZROWS, 64), jnp.float32),
            pltpu.VMEM_SHARED((N_PAD, 64), jnp.float32),
            pltpu.VMEM_SHARED((N_PAD, 64), jnp.float32),
            [pltpu.SemaphoreType.DMA] * NBUF,
            [pltpu.SemaphoreType.DMA] * NBUF,
            [pltpu.SemaphoreType.DMA] * NBUF,
        ],
        compiler_params=pltpu.CompilerParams(use_tc_tiling_on_sc=False),
    )(ptab.reshape(NC * gpc * N_PAD, 64), src2d, dst2d)


BT = 1000


def _t1_body(degp_ref, x_ref, dinv_ref, px_ref):
    deg = degp_ref[0, :, 0:1] + degp_ref[1, :, 0:1] + 1.0
    dinv = lax.rsqrt(deg)
    dinv_ref[:, :] = dinv
    p = x_ref[:, :] * dinv
    for g in range(2):
        px_ref[g, :, :] = p[:, 64 * g:64 * g + 64]


def _t2_body(s_ref, px_ref, dinv_ref, w1_ref, b1_ref, p1_ref):
    dinv = dinv_ref[:, :]
    h = b1_ref[:, :] + jnp.zeros((BT, 256), jnp.float32)
    for g in range(2):
        a = (s_ref[g, :, :] + px_ref[g, :, :]) * dinv
        h += jnp.dot(a, w1_ref[64 * g:64 * g + 64, :],
                     preferred_element_type=jnp.float32)
    p1 = jax.nn.relu(h) * dinv
    for g in range(4):
        p1_ref[g, :, :] = p1[:, 64 * g:64 * g + 64]


def _t3_body(s_ref, p1_ref, dinv_ref, w2_ref, b2_ref, out_ref):
    dinv = dinv_ref[:, :]
    o = b2_ref[:, :] + jnp.zeros((BT, 256), jnp.float32)
    for g in range(4):
        a = (s_ref[g, :, :] + p1_ref[g, :, :]) * dinv
        o += jnp.dot(a, w2_ref[64 * g:64 * g + 64, :],
                     preferred_element_type=jnp.float32)
    out_ref[:, :] = o


def _row_spec(shape2):
    return pl.BlockSpec((BT,) + shape2, lambda i: (i,) + (0,) * len(shape2))


def _grp_spec(g, w):
    return pl.BlockSpec((g, BT, w), lambda i: (0, i, 0))


def _full_spec(shape):
    return pl.BlockSpec(shape, lambda i: (0,) * len(shape))


def kernel(x, edge_index, W1, b1, W2, b2):
    npad = E_PAD - E
    src = jnp.concatenate([edge_index[0], jnp.zeros((npad,), jnp.int32)])
    dst = jnp.concatenate(
        [edge_index[1], jnp.full((npad,), DUMMY, jnp.int32)])
    src2d = src.reshape(E_PAD // CH, CH)
    dst2d = dst.reshape(E_PAD // CH, CH)

    degp = _sc_deg(dst2d)
    dinv, px = pl.pallas_call(
        _t1_body,
        grid=(N // BT,),
        in_specs=[_grp_spec(2, 16), _row_spec((128,))],
        out_specs=[_row_spec((1,)), _grp_spec(2, 64)],
        out_shape=(
            jax.ShapeDtypeStruct((N, 1), jnp.float32),
            jax.ShapeDtypeStruct((2, N_PAD, 64), jnp.float32),
        ),
    )(degp, x)

    sx = _sc_scatter(px, src2d, dst2d, gpc=1)
    p1 = pl.pallas_call(
        _t2_body,
        grid=(N // BT,),
        in_specs=[_grp_spec(2, 64), _grp_spec(2, 64), _row_spec((1,)),
                  _full_spec((128, 256)), _full_spec((1, 256))],
        out_specs=_grp_spec(4, 64),
        out_shape=jax.ShapeDtypeStruct((4, N_PAD, 64), jnp.float32),
    )(sx, px, dinv, W1, b1.reshape(1, 256))

    s1 = _sc_scatter(p1, src2d, dst2d, gpc=2)
    out = pl.pallas_call(
        _t3_body,
        grid=(N // BT,),
        in_specs=[_grp_spec(4, 64), _grp_spec(4, 64), _row_spec((1,)),
                  _full_spec((256, 256)), _full_spec((1, 256))],
        out_specs=_row_spec((256,)),
        out_shape=jax.ShapeDtypeStruct((N, 256), jnp.float32),
    )(s1, p1, dinv, W2, b2.reshape(1, 256))
    return out


# speedup vs baseline: 1.0132x; 1.0132x over previous
import functools

import jax
import jax.numpy as jnp
from jax import lax
from jax.experimental import pallas as pl
from jax.experimental.pallas import tpu as pltpu
from jax.experimental.pallas import tpu_sc as plsc

N = 10000
E = 320000
NC = 2
NS = 16
CH = 128
E_PAD = 327680
N_PAD = 10240
TILE_ROWS = N_PAD // NS
ZROWS = 64
DUMMY = N
NBUF = 5

_MESH = dict(core_axis_name="c", subcore_axis_name="s", num_cores=NC,
             num_subcores=NS)


def _zero_fill(buf, nrows, ncols):
    @pl.loop(0, nrows)
    def _(r):
        for j in range(ncols // 16):
            buf[r, pl.ds(j * 16, 16)] = jnp.zeros((16,), jnp.float32)


def _zero_acc(acc, s, zbuf):
    for k in range(TILE_ROWS // ZROWS):
        pltpu.sync_copy(zbuf, acc.at[pl.ds(s * TILE_ROWS + k * ZROWS, ZROWS)])


def _drain_acc(acc, out_hbm, g, s, zbuf):
    for k in range(TILE_ROWS // ZROWS):
        r0 = s * TILE_ROWS + k * ZROWS
        pltpu.sync_copy(acc.at[pl.ds(r0, ZROWS)], zbuf)
        pltpu.sync_copy(zbuf, out_hbm.at[g, pl.ds(r0, ZROWS)])


def _deg_body(dst2d, degp, dst_all, ones, zbuf, acc):
    c = lax.axis_index("c")
    s = lax.axis_index("s")
    rows = (E_PAD // NC // NS) // CH
    row0 = (c * NS + s) * rows

    pltpu.sync_copy(dst2d.at[pl.ds(row0, rows)], dst_all)

    @pl.loop(0, CH)
    def _(r):
        ones[r, :] = jnp.ones((16,), jnp.float32)

    _zero_fill(zbuf, ZROWS, 16)
    _zero_acc(acc, s, zbuf)
    plsc.subcore_barrier()

    @pl.loop(0, rows)
    def _(i):
        pltpu.sync_copy(ones, acc.at[dst_all.at[i]], add=True)

    plsc.subcore_barrier()
    _drain_acc(acc, degp, c, s, zbuf)


def _sc_deg(dst2d):
    return pl.kernel(
        _deg_body,
        out_type=jax.ShapeDtypeStruct((NC, N_PAD, 16), jnp.float32),
        mesh=plsc.VectorSubcoreMesh(**_MESH),
        scratch_types=[
            pltpu.VMEM((E_PAD // NC // NS // CH, CH), jnp.int32),
            pltpu.VMEM((CH, 16), jnp.float32),
            pltpu.VMEM((ZROWS, 16), jnp.float32),
            pltpu.VMEM_SHARED((N_PAD, 16), jnp.float32),
        ],
    )(dst2d)


STAGED = True


def _scatter_body(gpc, ptab, src2d, dst2d, out, sidx, didx, gbuf,
                  zbuf, stbl, acc, isems, gsems, ssems):
    c = lax.axis_index("c")
    s = lax.axis_index("s")
    rows = (E_PAD // NS) // CH
    row0 = s * rows

    def iload(i, b):
        pltpu.async_copy(src2d.at[row0 + i], sidx.at[b], isems[b])
        pltpu.async_copy(dst2d.at[row0 + i], didx.at[b], isems[b])

    def iwait(i, b, goff):
        pltpu.make_async_copy(src2d.at[row0 + i], sidx.at[b],
                              isems[b]).wait()
        pltpu.make_async_copy(dst2d.at[row0 + i], didx.at[b],
                              isems[b]).wait()
        if not STAGED:
            for j in range(CH // 16):
                v = sidx[b, pl.ds(j * 16, 16)]
                sidx[b, pl.ds(j * 16, 16)] = v + goff

    gsrc = stbl if STAGED else ptab

    def gstart(i, b):
        pltpu.async_copy(gsrc.at[sidx.at[b]], gbuf.at[b], gsems[b])

    def gwait(i, b):
        pltpu.make_async_copy(gsrc.at[sidx.at[b]], gbuf.at[b],
                              gsems[b]).wait()

    def sstart(i, b):
        pltpu.async_copy(gbuf.at[b], acc.at[didx.at[b]], ssems[b],
                         add=True)

    def swait(i, b):
        pltpu.make_async_copy(gbuf.at[b], acc.at[didx.at[b]],
                              ssems[b]).wait()

    for phase in range(gpc):
        g = c * gpc + phase
        goff = g * N_PAD
        if STAGED:
            pltpu.async_copy(
                ptab.at[pl.ds(goff + s * TILE_ROWS, TILE_ROWS)],
                stbl.at[pl.ds(s * TILE_ROWS, TILE_ROWS)], gsems[0])
        _zero_fill(zbuf, ZROWS, 64)
        nz = TILE_ROWS // ZROWS
        for k in range(nz):
            pltpu.async_copy(
                zbuf, acc.at[pl.ds(s * TILE_ROWS + k * ZROWS, ZROWS)],
                ssems[0])
        for k in range(nz):
            pltpu.make_async_copy(
                zbuf, acc.at[pl.ds(s * TILE_ROWS + k * ZROWS, ZROWS)],
                ssems[0]).wait()
        if STAGED:
            pltpu.make_async_copy(
                ptab.at[pl.ds(goff + s * TILE_ROWS, TILE_ROWS)],
                stbl.at[pl.ds(s * TILE_ROWS, TILE_ROWS)], gsems[0]).wait()
        plsc.subcore_barrier()

        for b in range(NBUF):
            iload(b, b)
        for b in range(NBUF):
            iwait(b, b, goff)
            gstart(b, b)

        @pl.loop(0, rows, step=NBUF)
        def _(i):
            for b in range(NBUF):
                gwait(i + b, b)
                sstart(i + b, b)
            for b in range(NBUF):
                swait(i + b, b)

                @pl.when(i + b + NBUF < rows)
                def _():
                    iload(i + b + NBUF, b)
                    iwait(i + b + NBUF, b, goff)
                    gstart(i + b + NBUF, b)

        plsc.subcore_barrier()
        for k in range(TILE_ROWS // CH):
            rk = s * TILE_ROWS + k * CH
            pltpu.async_copy(acc.at[pl.ds(rk, CH)], gbuf.at[k], gsems[k])
        for k in range(TILE_ROWS // CH):
            rk = s * TILE_ROWS + k * CH
            pltpu.make_async_copy(acc.at[pl.ds(rk, CH)], gbuf.at[k],
                                  gsems[k]).wait()
            pltpu.async_copy(gbuf.at[k], out.at[g, pl.ds(rk, CH)],
                             ssems[k])
        for k in range(TILE_ROWS // CH):
            rk = s * TILE_ROWS + k * CH
            pltpu.make_async_copy(gbuf.at[k], out.at[g, pl.ds(rk, CH)],
                                  ssems[k]).wait()
        if phase + 1 < gpc:
            plsc.subcore_barrier()


def _sc_scatter(ptab, src2d, dst2d, gpc):
    body = functools.partial(_scatter_body, gpc)
    return pl.kernel(
        body,
        out_type=jax.ShapeDtypeStruct((NC * gpc, N_PAD, 64), jnp.float32),
        mesh=plsc.VectorSubcoreMesh(**_MESH),
        scratch_types=[
            pltpu.VMEM((NBUF, CH), jnp.int32),
            pltpu.VMEM((NBUF, CH), jnp.int32),
            pltpu.VMEM((NBUF, CH, 64), jnp.float32),
            pltpu.VMEM((ZROWS, 64), jnp.float32),
            pltpu.VMEM_SHARED((N_PAD, 64), jnp.float32),
            pltpu.VMEM_SHARED((N_PAD, 64), jnp.float32),
            [pltpu.SemaphoreType.DMA] * NBUF,
            [pltpu.SemaphoreType.DMA] * NBUF,
            [pltpu.SemaphoreType.DMA] * NBUF,
        ],
        compiler_params=pltpu.CompilerParams(use_tc_tiling_on_sc=False),
    )(ptab.reshape(NC * gpc * N_PAD, 64), src2d, dst2d)


BT = 2000


def _t1_body(degp_ref, x_ref, dinv_ref, px_ref):
    deg = degp_ref[0, :, 0:1] + degp_ref[1, :, 0:1] + 1.0
    dinv = lax.rsqrt(deg)
    dinv_ref[:, :] = dinv
    p = x_ref[:, :] * dinv
    for g in range(2):
        px_ref[g, :, :] = p[:, 64 * g:64 * g + 64]


def _t2_body(s_ref, px_ref, dinv_ref, w1_ref, b1_ref, p1_ref):
    dinv = dinv_ref[:, :]
    h = b1_ref[:, :] + jnp.zeros((BT, 256), jnp.float32)
    for g in range(2):
        a = (s_ref[g, :, :] + px_ref[g, :, :]) * dinv
        h += jnp.dot(a, w1_ref[64 * g:64 * g + 64, :],
                     preferred_element_type=jnp.float32)
    p1 = jax.nn.relu(h) * dinv
    for g in range(4):
        p1_ref[g, :, :] = p1[:, 64 * g:64 * g + 64]


def _t3_body(s_ref, p1_ref, dinv_ref, w2_ref, b2_ref, out_ref):
    dinv = dinv_ref[:, :]
    o = b2_ref[:, :] + jnp.zeros((BT, 256), jnp.float32)
    for g in range(4):
        a = (s_ref[g, :, :] + p1_ref[g, :, :]) * dinv
        o += jnp.dot(a, w2_ref[64 * g:64 * g + 64, :],
                     preferred_element_type=jnp.float32)
    out_ref[:, :] = o


def _row_spec(shape2):
    return pl.BlockSpec((BT,) + shape2, lambda i: (i,) + (0,) * len(shape2))


def _grp_spec(g, w):
    return pl.BlockSpec((g, BT, w), lambda i: (0, i, 0))


def _full_spec(shape):
    return pl.BlockSpec(shape, lambda i: (0,) * len(shape))


def kernel(x, edge_index, W1, b1, W2, b2):
    npad = E_PAD - E
    src = jnp.concatenate([edge_index[0], jnp.zeros((npad,), jnp.int32)])
    dst = jnp.concatenate(
        [edge_index[1], jnp.full((npad,), DUMMY, jnp.int32)])
    src2d = src.reshape(E_PAD // CH, CH)
    dst2d = dst.reshape(E_PAD // CH, CH)

    degp = _sc_deg(dst2d)
    dinv, px = pl.pallas_call(
        _t1_body,
        grid=(N // BT,),
        in_specs=[_grp_spec(2, 16), _row_spec((128,))],
        out_specs=[_row_spec((1,)), _grp_spec(2, 64)],
        out_shape=(
            jax.ShapeDtypeStruct((N, 1), jnp.float32),
            jax.ShapeDtypeStruct((2, N_PAD, 64), jnp.float32),
        ),
    )(degp, x)

    sx = _sc_scatter(px, src2d, dst2d, gpc=1)
    p1 = pl.pallas_call(
        _t2_body,
        grid=(N // BT,),
        in_specs=[_grp_spec(2, 64), _grp_spec(2, 64), _row_spec((1,)),
                  _full_spec((128, 256)), _full_spec((1, 256))],
        out_specs=_grp_spec(4, 64),
        out_shape=jax.ShapeDtypeStruct((4, N_PAD, 64), jnp.float32),
    )(sx, px, dinv, W1, b1.reshape(1, 256))

    s1 = _sc_scatter(p1, src2d, dst2d, gpc=2)
    out = pl.pallas_call(
        _t3_body,
        grid=(N // BT,),
        in_specs=[_grp_spec(4, 64), _grp_spec(4, 64), _row_spec((1,)),
                  _full_spec((256, 256)), _full_spec((1, 256))],
        out_specs=_row_spec((256,)),
        out_shape=jax.ShapeDtypeStruct((N, 256), jnp.float32),
    )(s1, p1, dinv, W2, b2.reshape(1, 256))
    return out
